# baseline (speedup 1.0000x reference)
import functools

import jax
import jax.numpy as jnp
from jax import lax
from jax.experimental import pallas as pl
from jax.experimental.pallas import tpu as pltpu
from jax.experimental.pallas import tpu_sc as plsc

NC = 2
NS = 16
NW = NC * NS
L = 16

CHUNK = 1344
GROUPS = CHUNK // L

_LN2_HALF = 0.34657359027997264

_P0 = -2.7868866856258903
_P1 = 5.047106617093376
_P2 = -3.4927759978905084
_P3 = 1.5940691116760632
_P4 = -0.4049157020003179
_P5 = 0.043434338163993585

_SC_PARAMS = pltpu.CompilerParams(
    needs_layout_passes=False, use_tc_tiling_on_sc=False)


def _log_half(s):
  bits = plsc.bitcast(s, jnp.int32)
  ef = ((bits >> 23) - 127).astype(jnp.float32)
  m = plsc.bitcast((bits & 0x007FFFFF) | 0x3F800000, jnp.float32)
  m2 = m * m
  p01 = _P0 + m * _P1
  p23 = _P2 + m * _P3
  p45 = _P4 + m * _P5
  p = p01 + m2 * (p23 + m2 * p45)
  return (ef + p) * _LN2_HALF


def _make_body(m_edges):
  n_chunks = -(-m_edges // CHUNK)
  tail_ci = n_chunks - 1
  tail_base = m_edges - CHUNK
  tail_gstart = (n_chunks * CHUNK - m_edges) // L
  iters = -(-n_chunks // NW)

  def body(vt, idx0, idx1, le0, marg, out,
           ib0a, ib0b, ib0c, ib1a, ib1b, ib1c, lea, leb, lec,
           r0a, r0b, r1a, r1b, stage_v,
           semA0, semA1, semB0, semB1):
    ib0 = (ib0a, ib0b, ib0c)
    ib1 = (ib1a, ib1b, ib1c)
    le3 = (lea, leb, lec)
    r0 = (r0a, r0b)
    r1 = (r1a, r1b)
    semA = (semA0, semA1)
    semB = (semB0, semB1)

    wid = lax.axis_index("s") * NC + lax.axis_index("c")

    def chunk_base(k):
      ci = wid + k * NW
      clamped = jnp.minimum(ci, tail_ci)
      base = jnp.where(clamped == tail_ci, tail_base, clamped * CHUNK)
      gstart = jnp.where(ci > tail_ci, GROUPS,
                         jnp.where(ci == tail_ci, tail_gstart, 0))
      return base, gstart

    def start_idx(k, base):
      j = k % 3
      s = semA[k % 2]
      return (pltpu.async_copy(idx0.at[pl.ds(base, CHUNK)], ib0[j], s),
              pltpu.async_copy(idx1.at[pl.ds(base, CHUNK)], ib1[j], s),
              pltpu.async_copy(le0.at[pl.ds(base, CHUNK)], le3[j], s))

    def start_gather(k):
      j = k % 3
      b = k % 2
      s = semB[b]
      return (pltpu.async_copy(vt.at[ib0[j]], r0[b], s),
              pltpu.async_copy(vt.at[ib1[j]], r1[b], s))

    pltpu.sync_copy(marg, stage_v)
    margin_vec = stage_v[...]

    bases = []
    gstarts = []
    for k in range(iters):
      b, g = chunk_base(k)
      bases.append(b)
      gstarts.append(g)

    idx_cps = {}
    gat_cps = {}
    idx_cps[0] = start_idx(0, bases[0])
    if iters > 1:
      idx_cps[1] = start_idx(1, bases[1])
    for cp in idx_cps[0]:
      cp.wait()
    gat_cps[0] = start_gather(0)

    acc = jnp.zeros((L,), jnp.float32)
    for k in range(iters):
      for cp in gat_cps[k]:
        cp.wait()
      if k + 2 < iters:
        idx_cps[k + 2] = start_idx(k + 2, bases[k + 2])
      if k + 1 < iters:
        for cp in idx_cps[k + 1]:
          cp.wait()
        gat_cps[k + 1] = start_gather(k + 1)

      r0k = r0[k % 2]
      r1k = r1[k % 2]
      lek = le3[k % 3]

      def half_group(g, acc2, r0k, r1k, lek):
        e = g * L + lax.iota(jnp.int32, L)
        le = lek[pl.ds(g * L, L)]
        res = acc2
        for b in range(4):
          j0 = jnp.full((L,), 4 * b, jnp.int32)
          j1 = jnp.full((L,), 4 * b + 1, jnp.int32)
          j2 = jnp.full((L,), 4 * b + 2, jnp.int32)
          dx = plsc.load_gather(r1k, [e, j0]) - plsc.load_gather(r0k, [e, j0])
          dy = plsc.load_gather(r1k, [e, j1]) - plsc.load_gather(r0k, [e, j1])
          dz = plsc.load_gather(r1k, [e, j2]) - plsc.load_gather(r0k, [e, j2])
          s = dx * dx + dy * dy + dz * dz
          val = _log_half(s)
          val = jnp.where(s == 0.0, -jnp.inf, val)
          r = jnp.maximum(jnp.abs(val - le) - margin_vec, 0.0)
          res = res + r
        return res

      def group_body(g2, acc2, r0k=r0k, r1k=r1k, lek=lek):
        res = half_group(g2 * 2, acc2, r0k, r1k, lek)
        return half_group(g2 * 2 + 1, res, r0k, r1k, lek)

      acc = lax.fori_loop(gstarts[k] // 2, GROUPS // 2, group_body, acc)

    stage_v[...] = acc
    pltpu.sync_copy(stage_v, out.at[wid])

  return body


@functools.lru_cache(maxsize=None)
def _make_sc_call(m_edges):
  mesh = plsc.VectorSubcoreMesh(core_axis_name="c", subcore_axis_name="s")
  return pl.kernel(
      _make_body(m_edges),
      mesh=mesh,
      compiler_params=_SC_PARAMS,
      out_type=jax.ShapeDtypeStruct((NW, L), jnp.float32),
      scratch_types=[
          pltpu.VMEM((CHUNK,), jnp.int32),
          pltpu.VMEM((CHUNK,), jnp.int32),
          pltpu.VMEM((CHUNK,), jnp.int32),
          pltpu.VMEM((CHUNK,), jnp.int32),
          pltpu.VMEM((CHUNK,), jnp.int32),
          pltpu.VMEM((CHUNK,), jnp.int32),
          pltpu.VMEM((CHUNK,), jnp.float32),
          pltpu.VMEM((CHUNK,), jnp.float32),
          pltpu.VMEM((CHUNK,), jnp.float32),
          pltpu.VMEM((CHUNK, 16), jnp.float32),
          pltpu.VMEM((CHUNK, 16), jnp.float32),
          pltpu.VMEM((CHUNK, 16), jnp.float32),
          pltpu.VMEM((CHUNK, 16), jnp.float32),
          pltpu.VMEM((L,), jnp.float32),
          pltpu.SemaphoreType.DMA,
          pltpu.SemaphoreType.DMA,
          pltpu.SemaphoreType.DMA,
          pltpu.SemaphoreType.DMA,
      ],
  )


def kernel(verts, log_e0, all_edges, margin):
  B, N, _ = verts.shape
  M = all_edges.shape[0]

  idx = all_edges.astype(jnp.int32)
  vt = jnp.pad(verts, ((0, 0), (0, 0), (0, 1)))
  vt = jnp.transpose(vt, (1, 0, 2)).reshape(N, 4 * B)
  margv = jnp.full((L,), margin, jnp.float32)

  parts = _make_sc_call(M)(vt, idx[:, 0], idx[:, 1],
                           log_e0.astype(jnp.float32), margv)
  return jnp.sum(parts) / (B * M)


# speedup vs baseline: 1.2556x; 1.2556x over previous
import functools

import jax
import jax.numpy as jnp
from jax import lax
from jax.experimental import pallas as pl
from jax.experimental.pallas import tpu as pltpu
from jax.experimental.pallas import tpu_sc as plsc

NC = 2
NS = 16
NW = NC * NS
L = 16

CHUNK = 1344
GROUPS = CHUNK // L

_LN2_HALF = 0.34657359027997264

_P0 = -2.7868866856258903
_P1 = 5.047106617093376
_P2 = -3.4927759978905084
_P3 = 1.5940691116760632
_P4 = -0.4049157020003179
_P5 = 0.043434338163993585

_SC_PARAMS = pltpu.CompilerParams(
    needs_layout_passes=False, use_tc_tiling_on_sc=False)


_SQRT2 = 1.4142135623730951


def _log_half(s):
  bits = plsc.bitcast(s, jnp.int32)
  ex = (bits >> 23) - 127
  m = plsc.bitcast((bits & 0x007FFFFF) | 0x3F800000, jnp.float32)
  big = m > _SQRT2
  m = jnp.where(big, m * 0.5, m)
  ef = ex.astype(jnp.float32) + jnp.where(big, 1.0, 0.0)
  z = (m - 1.0) / (m + 1.0)
  z2 = z * z
  poly = 1.0 + z2 * (0.33333333 + z2 * (0.2 + z2 * 0.14285714))
  return ef * _LN2_HALF + z * poly


def _make_body(m_edges):
  n_chunks = -(-m_edges // CHUNK)
  tail_ci = n_chunks - 1
  tail_base = m_edges - CHUNK
  tail_gstart = (n_chunks * CHUNK - m_edges) // L
  iters = -(-n_chunks // NW)

  def body(vt, idx0, idx1, le0, marg, out,
           ib0a, ib0b, ib0c, ib1a, ib1b, ib1c, lea, leb, lec,
           r0a, r0b, r1a, r1b, stage_v,
           semA0, semA1, semB0, semB1):
    ib0 = (ib0a, ib0b, ib0c)
    ib1 = (ib1a, ib1b, ib1c)
    le3 = (lea, leb, lec)
    r0 = (r0a, r0b)
    r1 = (r1a, r1b)
    semA = (semA0, semA1)
    semB = (semB0, semB1)

    wid = lax.axis_index("s") * NC + lax.axis_index("c")

    def chunk_base(k):
      ci = wid + k * NW
      clamped = jnp.minimum(ci, tail_ci)
      base = jnp.where(clamped == tail_ci, tail_base, clamped * CHUNK)
      gstart = jnp.where(ci > tail_ci, GROUPS,
                         jnp.where(ci == tail_ci, tail_gstart, 0))
      return base, gstart

    def start_idx(k, base):
      j = k % 3
      s = semA[k % 2]
      return (pltpu.async_copy(idx0.at[pl.ds(base, CHUNK)], ib0[j], s),
              pltpu.async_copy(idx1.at[pl.ds(base, CHUNK)], ib1[j], s),
              pltpu.async_copy(le0.at[pl.ds(base, CHUNK)], le3[j], s))

    def start_gather(k):
      j = k % 3
      b = k % 2
      s = semB[b]
      return (pltpu.async_copy(vt.at[ib0[j]], r0[b], s),
              pltpu.async_copy(vt.at[ib1[j]], r1[b], s))

    pltpu.sync_copy(marg, stage_v)
    margin_vec = stage_v[...]

    bases = []
    gstarts = []
    for k in range(iters):
      b, g = chunk_base(k)
      bases.append(b)
      gstarts.append(g)

    idx_cps = {}
    gat_cps = {}
    idx_cps[0] = start_idx(0, bases[0])
    if iters > 1:
      idx_cps[1] = start_idx(1, bases[1])
    for cp in idx_cps[0]:
      cp.wait()
    gat_cps[0] = start_gather(0)

    acc = jnp.zeros((L,), jnp.float32)
    for k in range(iters):
      for cp in gat_cps[k]:
        cp.wait()
      if k + 2 < iters:
        idx_cps[k + 2] = start_idx(k + 2, bases[k + 2])
      if k + 1 < iters:
        for cp in idx_cps[k + 1]:
          cp.wait()
        gat_cps[k + 1] = start_gather(k + 1)

      r0k = r0[k % 2]
      r1k = r1[k % 2]
      lek = le3[k % 3]

      def half_group(g, acc2, r0k, r1k, lek):
        e = g * L + lax.iota(jnp.int32, L)
        le = lek[pl.ds(g * L, L)]
        res = acc2
        for b in range(4):
          j0 = jnp.full((L,), 4 * b, jnp.int32)
          j1 = jnp.full((L,), 4 * b + 1, jnp.int32)
          j2 = jnp.full((L,), 4 * b + 2, jnp.int32)
          dx = plsc.load_gather(r1k, [e, j0]) - plsc.load_gather(r0k, [e, j0])
          dy = plsc.load_gather(r1k, [e, j1]) - plsc.load_gather(r0k, [e, j1])
          dz = plsc.load_gather(r1k, [e, j2]) - plsc.load_gather(r0k, [e, j2])
          s = dx * dx + dy * dy + dz * dz
          val = _log_half(s)
          val = jnp.where(s == 0.0, -jnp.inf, val)
          r = jnp.maximum(jnp.abs(val - le) - margin_vec, 0.0)
          res = res + r
        return res

      def group_body(g, acc2, r0k=r0k, r1k=r1k, lek=lek):
        return half_group(g, acc2, r0k, r1k, lek)

      acc = lax.fori_loop(gstarts[k], GROUPS, group_body, acc)

    stage_v[...] = acc
    pltpu.sync_copy(stage_v, out.at[wid])

  return body


@functools.lru_cache(maxsize=None)
def _make_sc_call(m_edges):
  mesh = plsc.VectorSubcoreMesh(core_axis_name="c", subcore_axis_name="s")
  return pl.kernel(
      _make_body(m_edges),
      mesh=mesh,
      compiler_params=_SC_PARAMS,
      out_type=jax.ShapeDtypeStruct((NW, L), jnp.float32),
      scratch_types=[
          pltpu.VMEM((CHUNK,), jnp.int32),
          pltpu.VMEM((CHUNK,), jnp.int32),
          pltpu.VMEM((CHUNK,), jnp.int32),
          pltpu.VMEM((CHUNK,), jnp.int32),
          pltpu.VMEM((CHUNK,), jnp.int32),
          pltpu.VMEM((CHUNK,), jnp.int32),
          pltpu.VMEM((CHUNK,), jnp.float32),
          pltpu.VMEM((CHUNK,), jnp.float32),
          pltpu.VMEM((CHUNK,), jnp.float32),
          pltpu.VMEM((CHUNK, 16), jnp.float32),
          pltpu.VMEM((CHUNK, 16), jnp.float32),
          pltpu.VMEM((CHUNK, 16), jnp.float32),
          pltpu.VMEM((CHUNK, 16), jnp.float32),
          pltpu.VMEM((L,), jnp.float32),
          pltpu.SemaphoreType.DMA,
          pltpu.SemaphoreType.DMA,
          pltpu.SemaphoreType.DMA,
          pltpu.SemaphoreType.DMA,
      ],
  )


def kernel(verts, log_e0, all_edges, margin):
  B, N, _ = verts.shape
  M = all_edges.shape[0]

  idx = all_edges.astype(jnp.int32)
  vt = jnp.pad(verts, ((0, 0), (0, 0), (0, 1)))
  vt = jnp.transpose(vt, (1, 0, 2)).reshape(N, 4 * B)
  margv = jnp.full((L,), margin, jnp.float32)

  parts = _make_sc_call(M)(vt, idx[:, 0], idx[:, 1],
                           log_e0.astype(jnp.float32), margv)
  return jnp.sum(parts) / (B * M)
